# initial kernel scaffold (unmeasured)
import jax
import jax.numpy as jnp
from jax import lax
from jax.experimental import pallas as pl
from jax.experimental.pallas import tpu as pltpu


def kernel(
    x,
):
    def body(*refs):
        pass

    out_shape = jax.ShapeDtypeStruct(..., jnp.float32)
    return pl.pallas_call(body, out_shape=out_shape)(...)


# baseline (device time: 13385 ns/iter reference)
import jax
import jax.numpy as jnp
from jax import lax
from jax.experimental import pallas as pl
from jax.experimental.pallas import tpu as pltpu

Y = 4
K = 8

_NEG = float(jnp.finfo(jnp.float32).min)


def _topk_desc(vals, k):
    cols = []
    for _ in range(k):
        mx = jnp.max(vals, axis=1, keepdims=True)
        cols.append(mx)
        vals = jnp.where(vals == mx, _NEG, vals)
    return jnp.concatenate(cols, axis=1)


def kernel(x):
    m, n = x.shape

    def body(x_ref, out_ref, comm_ref, send_sems, recv_sems):
        my_x = lax.axis_index("x")
        my_y = lax.axis_index("y")
        my_z = lax.axis_index("z")

        barrier_sem = pltpu.get_barrier_semaphore()
        for o in range(1, Y):
            ty = lax.rem(my_y + o, Y)
            pl.semaphore_signal(
                barrier_sem,
                inc=1,
                device_id=(my_x, ty, my_z),
                device_id_type=pl.DeviceIdType.MESH,
            )
        pl.semaphore_wait(barrier_sem, Y - 1)

        top = _topk_desc(x_ref[:, :].astype(jnp.float32), K)
        comm_ref[0] = top

        sends = []
        for o in range(1, Y):
            ty = lax.rem(my_y + o, Y)
            rdma = pltpu.make_async_remote_copy(
                src_ref=comm_ref.at[0],
                dst_ref=comm_ref.at[Y - o],
                send_sem=send_sems.at[o],
                recv_sem=recv_sems.at[Y - o],
                device_id=(my_x, ty, my_z),
                device_id_type=pl.DeviceIdType.MESH,
            )
            rdma.start()
            sends.append(rdma)

        for j in range(1, Y):
            recv = pltpu.make_async_remote_copy(
                src_ref=comm_ref.at[0],
                dst_ref=comm_ref.at[j],
                send_sem=send_sems.at[j],
                recv_sem=recv_sems.at[j],
                device_id=(my_x, my_y, my_z),
                device_id_type=pl.DeviceIdType.MESH,
            )
            recv.wait_recv()
        for rdma in sends:
            rdma.wait_send()

        cand = jnp.concatenate([comm_ref[j] for j in range(Y)], axis=1)
        out_ref[:, :] = _topk_desc(cand, K)

    return pl.pallas_call(
        body,
        out_shape=jax.ShapeDtypeStruct((m, K), jnp.float32),
        in_specs=[pl.BlockSpec(memory_space=pltpu.VMEM)],
        out_specs=pl.BlockSpec(memory_space=pltpu.VMEM),
        scratch_shapes=[
            pltpu.VMEM((Y, m, K), jnp.float32),
            pltpu.SemaphoreType.DMA((Y,)),
            pltpu.SemaphoreType.DMA((Y,)),
        ],
        compiler_params=pltpu.CompilerParams(collective_id=0),
    )(x)
